# baseline (speedup 1.0000x reference)
import functools

import jax
import jax.numpy as jnp
from jax import lax
from jax.experimental import pallas as pl
from jax.experimental.pallas import tpu as pltpu
from jax.experimental.pallas import tpu_sc as plsc

U_SIZE = 1000000
I_SIZE = 100000
DIM = 64
BATCH = 16384
LINE = 2 * DIM

NUM_CORES = 2
NUM_SUBCORES = 16
NUM_WORKERS = NUM_CORES * NUM_SUBCORES
ROWS_PER_WORKER = BATCH // NUM_WORKERS
CHUNK = 128
NUM_CHUNKS = ROWS_PER_WORKER // CHUNK
GROUPS = CHUNK // 16


def _make_kernel():
  mesh = plsc.VectorSubcoreMesh(core_axis_name="c", subcore_axis_name="s")

  @functools.partial(
      pl.kernel,
      mesh=mesh,
      out_type=jax.ShapeDtypeStruct((BATCH,), jnp.float32),
      compiler_params=pltpu.CompilerParams(
          needs_layout_passes=False, use_tc_tiling_on_sc=True),
      scratch_types=[
          pltpu.VMEM((ROWS_PER_WORKER,), jnp.int32),
          pltpu.VMEM((ROWS_PER_WORKER,), jnp.int32),
          pltpu.VMEM((CHUNK,), jnp.int32),
          pltpu.VMEM((CHUNK,), jnp.int32),
          pltpu.VMEM((CHUNK, LINE), jnp.float32),
          pltpu.VMEM((CHUNK, LINE), jnp.float32),
          pltpu.VMEM((ROWS_PER_WORKER,), jnp.float32),
          pltpu.SemaphoreType.DMA,
      ],
  )
  def k(users_hbm, items_hbm, user_lin_hbm, item_lin_hbm, out_hbm,
        uidx_v, iidx_v, uq_v, iq_v, ulin_v, ilin_v, out_v, sem):
    wid = lax.axis_index("s") * NUM_CORES + lax.axis_index("c")
    base = wid * ROWS_PER_WORKER

    pltpu.sync_copy(users_hbm.at[pl.ds(base, ROWS_PER_WORKER)], uidx_v)
    pltpu.sync_copy(items_hbm.at[pl.ds(base, ROWS_PER_WORKER)], iidx_v)

    lanes = lax.iota(jnp.int32, 16)

    def chunk_body(c, _):
      row0 = c * CHUNK
      for t in range(CHUNK // 16):
        uq_v[pl.ds(t * 16, 16)] = uidx_v[pl.ds(row0 + t * 16, 16)] >> 1
        iq_v[pl.ds(t * 16, 16)] = iidx_v[pl.ds(row0 + t * 16, 16)] >> 1
      cu = pltpu.async_copy(user_lin_hbm.at[uq_v], ulin_v, sem)
      ci = pltpu.async_copy(item_lin_hbm.at[iq_v], ilin_v, sem)
      cu.wait()
      ci.wait()

      def group_body(g, _g):
        j_vec = g * 16 + lanes
        ubase = (uidx_v[pl.ds(row0 + g * 16, 16)] & 1) * DIM
        ibase = (iidx_v[pl.ds(row0 + g * 16, 16)] & 1) * DIM
        acc = jnp.zeros((16,), jnp.float32)
        for d in range(DIM):
          col = (lanes + d) & (DIM - 1)
          ug = plsc.load_gather(ulin_v, [j_vec, ubase + col])
          ig = plsc.load_gather(ilin_v, [j_vec, ibase + col])
          acc = acc + ug * ig
        out_v[pl.ds(row0 + g * 16, 16)] = acc
        return _g

      lax.fori_loop(0, GROUPS, group_body, 0, unroll=False)
      return _

    lax.fori_loop(0, NUM_CHUNKS, chunk_body, 0, unroll=False)

    pltpu.sync_copy(out_v, out_hbm.at[pl.ds(base, ROWS_PER_WORKER)])

  return k


_kernel_call = _make_kernel()


@jax.jit
def kernel(users, items, user_emb, item_emb):
  user_lin = user_emb.reshape(U_SIZE // 2, LINE)
  item_lin = item_emb.reshape(I_SIZE // 2, LINE)
  return _kernel_call(users, items, user_lin, item_lin)
